# baseline (speedup 1.0000x reference)
import functools
import math

import jax
import jax.numpy as jnp
from jax import lax
from jax.experimental import pallas as pl
from jax.experimental.pallas import tpu as pltpu
from jax.experimental.pallas import tpu_sc as plsc

N_ROWS = 200
D = 128
B_TOTAL = 4096 * 200
NW = 32
BPW = B_TOTAL // NW
CH = 128
KCH = 1
SC_ROWS = KCH * CH
NSC = BPW // SC_ROWS
NCHUNK = B_TOTAL // CH
NBUF = 3


def _proj_body(emb_ref, w_ref, b_ref, out_ref):
    out_ref[...] = (
        jnp.dot(emb_ref[...], w_ref[...].T, preferred_element_type=jnp.float32)
        + b_ref[...]
    )


def _project_table(emb_weight, lin_w, lin_b):
    return pl.pallas_call(
        _proj_body,
        out_shape=jax.ShapeDtypeStruct((N_ROWS, D), jnp.float32),
    )(emb_weight, lin_w, lin_b.reshape(1, D))


_mesh = plsc.VectorSubcoreMesh(core_axis_name="c", subcore_axis_name="s")


@functools.partial(
    pl.kernel,
    mesh=_mesh,
    out_type=jax.ShapeDtypeStruct((B_TOTAL, D), jnp.float32),
    scratch_types=(
        [pltpu.VMEM((KCH, CH), jnp.int32) for _ in range(NBUF)]
        + [pltpu.VMEM((SC_ROWS, D), jnp.float32) for _ in range(NBUF)]
        + [pltpu.VMEM_SHARED((N_ROWS, D), jnp.float32)]
        + [pltpu.SemaphoreType.DMA for _ in range(3 * NBUF)]
    ),
)
def _gather_kernel(table_hbm, idx_hbm, out_hbm,
                   idx0, idx1, idx2, rows0, rows1, rows2, table_v,
                   semg0, semg1, semg2, semw0, semw1, semw2,
                   semi0, semi1, semi2):
    wid = lax.axis_index("s") * 2 + lax.axis_index("c")
    base = wid * BPW
    cbase = wid * (BPW // CH)

    idx = (idx0, idx1, idx2)
    rows = (rows0, rows1, rows2)
    semg = (semg0, semg1, semg2)
    semw = (semw0, semw1, semw2)
    semi = (semi0, semi1, semi2)

    def fire_gather(i, b):
        for k in range(KCH):
            pltpu.async_copy(
                table_v.at[idx[b].at[k]],
                rows[b].at[pl.ds(k * CH, CH)],
                semg[b],
            )

    def drain_gather(b):
        pltpu.make_async_copy(
            out_hbm.at[pl.ds(0, SC_ROWS)], rows[b], semg[b]
        ).wait()

    def fire_write(i, b):
        pltpu.async_copy(
            rows[b], out_hbm.at[pl.ds(base + i * SC_ROWS, SC_ROWS)], semw[b]
        )

    def drain_write(b):
        pltpu.make_async_copy(
            rows[b], out_hbm.at[pl.ds(0, SC_ROWS)], semw[b]
        ).wait()

    def fire_idx(i, b):
        pltpu.async_copy(
            idx_hbm.at[pl.ds(cbase + i * KCH, KCH)], idx[b], semi[b]
        )

    def drain_idx(b):
        pltpu.make_async_copy(
            idx_hbm.at[pl.ds(0, KCH)], idx[b], semi[b]
        ).wait()

    def phase(i, b):
        nb = (b + 1) % NBUF

        @pl.when(i + 1 < NSC)
        def _():
            @pl.when(i >= NBUF - 1)
            def _():
                drain_write(nb)
            drain_idx(nb)
            fire_gather(i + 1, nb)

        drain_gather(b)
        fire_write(i, b)

        @pl.when(i + NBUF < NSC)
        def _():
            fire_idx(i + NBUF, b)

    @pl.when(lax.axis_index("s") == 0)
    def _():
        pltpu.sync_copy(table_hbm, table_v)

    plsc.subcore_barrier()

    pltpu.sync_copy(idx_hbm.at[pl.ds(cbase, KCH)], idx0)
    fire_idx(1, 1)
    fire_idx(2, 2)
    fire_gather(0, 0)

    def body(q, carry):
        phase(3 * q + 0, 0)
        phase(3 * q + 1, 1)
        phase(3 * q + 2, 2)
        return carry

    lax.fori_loop(0, NSC // 3, body, 0)

    for i in range(3 * (NSC // 3), NSC):
        phase(i, i % NBUF)
    drain_write(0)
    drain_write(1)
    drain_write(2)


def kernel(t, emb_weight, lin_w, lin_b):
    proj = _project_table(emb_weight, lin_w, lin_b)
    idx = t.reshape(NCHUNK, CH)
    out = _gather_kernel(proj, idx)
    return out.reshape(t.shape[0], t.shape[1], D)


# speedup vs baseline: 1.0170x; 1.0170x over previous
import functools
import math

import jax
import jax.numpy as jnp
from jax import lax
from jax.experimental import pallas as pl
from jax.experimental.pallas import tpu as pltpu
from jax.experimental.pallas import tpu_sc as plsc

N_ROWS = 200
D = 128
B_TOTAL = 4096 * 200
NW = 32
BPW = B_TOTAL // NW
CH = 128
KCH = 2
SC_ROWS = KCH * CH
NSC = BPW // SC_ROWS
NCHUNK = B_TOTAL // CH
NBUF = 3


def _proj_body(emb_ref, w_ref, b_ref, out_ref):
    out_ref[...] = (
        jnp.dot(emb_ref[...], w_ref[...].T, preferred_element_type=jnp.float32)
        + b_ref[...]
    )


def _project_table(emb_weight, lin_w, lin_b):
    return pl.pallas_call(
        _proj_body,
        out_shape=jax.ShapeDtypeStruct((N_ROWS, D), jnp.float32),
    )(emb_weight, lin_w, lin_b.reshape(1, D))


_mesh = plsc.VectorSubcoreMesh(core_axis_name="c", subcore_axis_name="s")


@functools.partial(
    pl.kernel,
    mesh=_mesh,
    out_type=jax.ShapeDtypeStruct((B_TOTAL, D), jnp.float32),
    scratch_types=(
        [pltpu.VMEM((KCH, CH), jnp.int32) for _ in range(NBUF)]
        + [pltpu.VMEM((SC_ROWS, D), jnp.float32) for _ in range(NBUF)]
        + [pltpu.VMEM_SHARED((N_ROWS, D), jnp.float32)]
        + [pltpu.SemaphoreType.DMA for _ in range(3 * NBUF)]
    ),
)
def _gather_kernel(table_hbm, idx_hbm, out_hbm,
                   idx0, idx1, idx2, rows0, rows1, rows2, table_v,
                   semg0, semg1, semg2, semw0, semw1, semw2,
                   semi0, semi1, semi2):
    wid = lax.axis_index("s") * 2 + lax.axis_index("c")
    base = wid * BPW
    cbase = wid * (BPW // CH)

    idx = (idx0, idx1, idx2)
    rows = (rows0, rows1, rows2)
    semg = (semg0, semg1, semg2)
    semw = (semw0, semw1, semw2)
    semi = (semi0, semi1, semi2)

    def fire_gather(i, b):
        for k in range(KCH):
            pltpu.async_copy(
                table_v.at[idx[b].at[k]],
                rows[b].at[pl.ds(k * CH, CH)],
                semg[b],
            )

    def drain_gather(b):
        pltpu.make_async_copy(
            out_hbm.at[pl.ds(0, SC_ROWS)], rows[b], semg[b]
        ).wait()

    def fire_write(i, b):
        pltpu.async_copy(
            rows[b], out_hbm.at[pl.ds(base + i * SC_ROWS, SC_ROWS)], semw[b]
        )

    def drain_write(b):
        pltpu.make_async_copy(
            rows[b], out_hbm.at[pl.ds(0, SC_ROWS)], semw[b]
        ).wait()

    def fire_idx(i, b):
        pltpu.async_copy(
            idx_hbm.at[pl.ds(cbase + i * KCH, KCH)], idx[b], semi[b]
        )

    def drain_idx(b):
        pltpu.make_async_copy(
            idx_hbm.at[pl.ds(0, KCH)], idx[b], semi[b]
        ).wait()

    def phase(i, b):
        nb = (b + 1) % NBUF

        @pl.when(i + 1 < NSC)
        def _():
            @pl.when(i >= NBUF - 1)
            def _():
                drain_write(nb)
            drain_idx(nb)
            fire_gather(i + 1, nb)

        drain_gather(b)
        fire_write(i, b)

        @pl.when(i + NBUF < NSC)
        def _():
            fire_idx(i + NBUF, b)

    @pl.when(lax.axis_index("s") == 0)
    def _():
        pltpu.sync_copy(table_hbm, table_v)

    plsc.subcore_barrier()

    pltpu.sync_copy(idx_hbm.at[pl.ds(cbase, KCH)], idx0)
    fire_idx(1, 1)
    fire_idx(2, 2)
    fire_gather(0, 0)

    def body(q, carry):
        phase(3 * q + 0, 0)
        phase(3 * q + 1, 1)
        phase(3 * q + 2, 2)
        return carry

    lax.fori_loop(0, NSC // 3, body, 0)

    for i in range(3 * (NSC // 3), NSC):
        phase(i, i % NBUF)
    drain_write(0)
    drain_write(1)
    drain_write(2)


def kernel(t, emb_weight, lin_w, lin_b):
    proj = _project_table(emb_weight, lin_w, lin_b)
    idx = t.reshape(NCHUNK, CH)
    out = _gather_kernel(proj, idx)
    return out.reshape(t.shape[0], t.shape[1], D)


# speedup vs baseline: 1.0203x; 1.0033x over previous
import functools
import math

import jax
import jax.numpy as jnp
from jax import lax
from jax.experimental import pallas as pl
from jax.experimental.pallas import tpu as pltpu
from jax.experimental.pallas import tpu_sc as plsc

N_ROWS = 200
D = 128
B_TOTAL = 4096 * 200
NW = 32
BPW = B_TOTAL // NW
CH = 128
KCH = 2
SC_ROWS = KCH * CH
NSC = BPW // SC_ROWS
NCHUNK = B_TOTAL // CH
NBUF = 3


def _proj_body(emb_ref, w_ref, b_ref, out_ref):
    out_ref[...] = (
        jnp.dot(emb_ref[...], w_ref[...].T, preferred_element_type=jnp.float32)
        + b_ref[...]
    )


def _project_table(emb_weight, lin_w, lin_b):
    return pl.pallas_call(
        _proj_body,
        out_shape=jax.ShapeDtypeStruct((N_ROWS, D), jnp.float32),
    )(emb_weight, lin_w, lin_b.reshape(1, D))


_mesh = plsc.VectorSubcoreMesh(core_axis_name="c", subcore_axis_name="s")


@functools.partial(
    pl.kernel,
    mesh=_mesh,
    out_type=jax.ShapeDtypeStruct((B_TOTAL, D), jnp.float32),
    scratch_types=(
        [pltpu.VMEM((KCH, CH), jnp.int32) for _ in range(NBUF)]
        + [pltpu.VMEM((SC_ROWS, D), jnp.float32) for _ in range(NBUF)]
        + [pltpu.VMEM_SHARED((N_ROWS, D), jnp.float32)]
        + [pltpu.SemaphoreType.DMA for _ in range(3 * NBUF)]
    ),
)
def _gather_kernel(table_hbm, idx_hbm, out_hbm,
                   idx0, idx1, idx2, rows0, rows1, rows2, table_v,
                   semg0, semg1, semg2, semw0, semw1, semw2,
                   semi0, semi1, semi2):
    wid = lax.axis_index("c") * 16 + lax.axis_index("s")
    base = wid * BPW
    cbase = wid * (BPW // CH)

    idx = (idx0, idx1, idx2)
    rows = (rows0, rows1, rows2)
    semg = (semg0, semg1, semg2)
    semw = (semw0, semw1, semw2)
    semi = (semi0, semi1, semi2)

    def fire_gather(i, b):
        for k in range(KCH):
            pltpu.async_copy(
                table_v.at[idx[b].at[k]],
                rows[b].at[pl.ds(k * CH, CH)],
                semg[b],
            )

    def drain_gather(b):
        pltpu.make_async_copy(
            out_hbm.at[pl.ds(0, SC_ROWS)], rows[b], semg[b]
        ).wait()

    def fire_write(i, b):
        pltpu.async_copy(
            rows[b], out_hbm.at[pl.ds(base + i * SC_ROWS, SC_ROWS)], semw[b]
        )

    def drain_write(b):
        pltpu.make_async_copy(
            rows[b], out_hbm.at[pl.ds(0, SC_ROWS)], semw[b]
        ).wait()

    def fire_idx(i, b):
        pltpu.async_copy(
            idx_hbm.at[pl.ds(cbase + i * KCH, KCH)], idx[b], semi[b]
        )

    def drain_idx(b):
        pltpu.make_async_copy(
            idx_hbm.at[pl.ds(0, KCH)], idx[b], semi[b]
        ).wait()

    def phase(i, b):
        nb = (b + 1) % NBUF

        @pl.when(i + 1 < NSC)
        def _():
            @pl.when(i >= NBUF - 1)
            def _():
                drain_write(nb)
            drain_idx(nb)
            fire_gather(i + 1, nb)

        drain_gather(b)
        fire_write(i, b)

        @pl.when(i + NBUF < NSC)
        def _():
            fire_idx(i + NBUF, b)

    @pl.when(lax.axis_index("s") == 0)
    def _():
        pltpu.sync_copy(table_hbm, table_v)

    plsc.subcore_barrier()

    pltpu.sync_copy(idx_hbm.at[pl.ds(cbase, KCH)], idx0)
    fire_idx(1, 1)
    fire_idx(2, 2)
    fire_gather(0, 0)

    def body(q, carry):
        phase(3 * q + 0, 0)
        phase(3 * q + 1, 1)
        phase(3 * q + 2, 2)
        return carry

    lax.fori_loop(0, NSC // 3, body, 0)

    for i in range(3 * (NSC // 3), NSC):
        phase(i, i % NBUF)
    drain_write(0)
    drain_write(1)
    drain_write(2)


def kernel(t, emb_weight, lin_w, lin_b):
    proj = _project_table(emb_weight, lin_w, lin_b)
    idx = t.reshape(NCHUNK, CH)
    out = _gather_kernel(proj, idx)
    return out.reshape(t.shape[0], t.shape[1], D)
